# Initial kernel scaffold; baseline (speedup 1.0000x reference)
import jax
import jax.numpy as jnp
from jax.experimental import pallas as pl


def kernel(x, edge_index, W1, b1, W2, b2):
    raise NotImplementedError("write your pallas kernel here")


# speedup vs baseline: 9.3218x; 9.3218x over previous
import functools

import jax
import jax.numpy as jnp
from jax import lax
from jax.experimental import pallas as pl
from jax.experimental.pallas import tpu as pltpu
from jax.experimental.pallas import tpu_sc as plsc

NC = 2
NS = 16
NW = NC * NS
LANE = 16
K = 128

_Z16 = None


def _zero_rows(ref, nrows, ncols):
  zv = jnp.zeros((LANE,), jnp.float32)

  def row(i, _):
    def col(c, __):
      ref[i, pl.ds(c * LANE, LANE)] = zv
      return 0
    lax.fori_loop(0, ncols // LANE, col, 0)
    return 0

  lax.fori_loop(0, nrows, row, 0)


def _deg_body(n_pad, rpt, n_chunks, dst_hbm, out_hbm, dst_v, ones_v, zbuf_v,
              deg_sh, sem):
  c = lax.axis_index("c")
  s = lax.axis_index("s")
  wid = c * NS + s

  pltpu.sync_copy(dst_hbm.at[pl.ds(wid * n_chunks, n_chunks)], dst_v)

  ov = jnp.ones((LANE,), jnp.float32)
  zv = jnp.zeros((LANE,), jnp.float32)

  def fill(i, _):
    ones_v[pl.ds(i * LANE, LANE)] = ov
    return 0
  lax.fori_loop(0, K // LANE, fill, 0)

  def zfill(i, _):
    zbuf_v[pl.ds(i * LANE, LANE)] = zv
    return 0
  lax.fori_loop(0, rpt // LANE, zfill, 0)

  pltpu.sync_copy(zbuf_v, deg_sh.at[pl.ds(s * rpt, rpt)])
  plsc.subcore_barrier()

  def chunk(j, _):
    pltpu.sync_copy(ones_v, deg_sh.at[dst_v.at[j]], add=True)
    return 0
  lax.fori_loop(0, n_chunks, chunk, 0)

  plsc.subcore_barrier()

  pltpu.sync_copy(deg_sh.at[pl.ds(s * rpt, rpt)],
                  out_hbm.at[pl.ds(c * n_pad + s * rpt, rpt)])


def _agg_body(n_pad, d, rpt, n_chunks, g_hbm, src_hbm, dst_hbm, out_hbm,
              src_v, dst_v, rows_v, acc_sh, sem):
  c = lax.axis_index("c")
  s = lax.axis_index("s")
  wid = c * NS + s

  pltpu.sync_copy(src_hbm.at[pl.ds(wid * n_chunks, n_chunks)], src_v)
  pltpu.sync_copy(dst_hbm.at[pl.ds(wid * n_chunks, n_chunks)], dst_v)

  _zero_rows(rows_v, K, d)
  for j in range(rpt // K):
    pltpu.sync_copy(rows_v, acc_sh.at[pl.ds(s * rpt + j * K, K)])
  plsc.subcore_barrier()

  def chunk(j, _):
    pltpu.async_copy(g_hbm.at[src_v.at[j]], rows_v, sem).wait()
    pltpu.sync_copy(rows_v, acc_sh.at[dst_v.at[j]], add=True)
    return 0
  lax.fori_loop(0, n_chunks, chunk, 0)

  plsc.subcore_barrier()

  for j in range(rpt // K):
    pltpu.sync_copy(acc_sh.at[pl.ds(s * rpt + j * K, K)],
                    out_hbm.at[pl.ds(c * n_pad + s * rpt + j * K, K)])


@functools.lru_cache(maxsize=None)
def _make_deg(n_pad, n_chunks):
  rpt = n_pad // NS
  mesh = plsc.VectorSubcoreMesh(core_axis_name="c", subcore_axis_name="s",
                                num_cores=NC, num_subcores=NS)
  return pl.kernel(
      functools.partial(_deg_body, n_pad, rpt, n_chunks),
      out_type=jax.ShapeDtypeStruct((NC * n_pad,), jnp.float32),
      mesh=mesh,
      scratch_types=[
          pltpu.VMEM((n_chunks, K), jnp.int32),
          pltpu.VMEM((K,), jnp.float32),
          pltpu.VMEM((rpt,), jnp.float32),
          pltpu.VMEM_SHARED((n_pad,), jnp.float32),
          pltpu.SemaphoreType.DMA,
      ],
  )


@functools.lru_cache(maxsize=None)
def _make_agg(n_pad, d, n_chunks):
  rpt = n_pad // NS
  mesh = plsc.VectorSubcoreMesh(core_axis_name="c", subcore_axis_name="s",
                                num_cores=NC, num_subcores=NS)
  return pl.kernel(
      functools.partial(_agg_body, n_pad, d, rpt, n_chunks),
      out_type=jax.ShapeDtypeStruct((NC * n_pad, d), jnp.float32),
      mesh=mesh,
      scratch_types=[
          pltpu.VMEM((n_chunks, K), jnp.int32),
          pltpu.VMEM((n_chunks, K), jnp.int32),
          pltpu.VMEM((K, d), jnp.float32),
          pltpu.VMEM_SHARED((n_pad, d), jnp.float32),
          pltpu.SemaphoreType.DMA,
      ],
  )


def _tc1_body(x_ref, w_ref, d0_ref, d1_ref, g_ref):
  dinv = lax.rsqrt(d0_ref[...] + d1_ref[...] + 1.0)
  g_ref[...] = jnp.dot(x_ref[...], w_ref[...],
                       preferred_element_type=jnp.float32) * dinv


def _tc2_body(p0_ref, p1_ref, g_ref, b_ref, w_ref, d0_ref, d1_ref, o_ref):
  dinv = lax.rsqrt(d0_ref[...] + d1_ref[...] + 1.0)
  z = jnp.maximum(
      dinv * (p0_ref[...] + p1_ref[...] + g_ref[...]) + b_ref[...], 0.0)
  o_ref[...] = jnp.dot(z, w_ref[...], preferred_element_type=jnp.float32) * dinv


def _tc3_body(p0_ref, p1_ref, g_ref, b_ref, d0_ref, d1_ref, o_ref):
  dinv = lax.rsqrt(d0_ref[...] + d1_ref[...] + 1.0)
  o_ref[...] = jnp.maximum(
      dinv * (p0_ref[...] + p1_ref[...] + g_ref[...]) + b_ref[...], 0.0)


def _row_spec(bn, d):
  return pl.BlockSpec((bn, d), lambda i: (i, 0))


def _full_spec(shape):
  return pl.BlockSpec(shape, lambda i: tuple(0 for _ in shape))


def kernel(x, edge_index, W1, b1, W2, b2):
  n, d_in = x.shape
  hid = W1.shape[1]
  e = edge_index.shape[1]

  src = edge_index[0].astype(jnp.int32)
  dst = edge_index[1].astype(jnp.int32)

  e_pad = -(-e // (NW * K * 8)) * (NW * K * 8)
  pad = e_pad - e
  if pad:
    src = jnp.concatenate([src, jnp.zeros((pad,), jnp.int32)])
    dst = jnp.concatenate([dst, jnp.full((pad,), n, jnp.int32)])
  n_chunks = e_pad // (NW * K)
  src2d = src.reshape(NW * n_chunks, K)
  dst2d = dst.reshape(NW * n_chunks, K)

  n_pad = -(-(n + 1) // (NS * K)) * (NS * K)

  deg = _make_deg(n_pad, n_chunks)(dst2d).reshape(NC, n_pad)
  d0 = deg[0, :n][:, None]
  d1 = deg[1, :n][:, None]

  bn = 2000 if n % 2000 == 0 else n
  grid = (n // bn,)
  b1r = b1.reshape(1, hid)
  b2r = b2.reshape(1, hid)

  g1 = pl.pallas_call(
      _tc1_body,
      grid=grid,
      in_specs=[_row_spec(bn, d_in), _full_spec((d_in, hid)),
                _row_spec(bn, 1), _row_spec(bn, 1)],
      out_specs=_row_spec(bn, hid),
      out_shape=jax.ShapeDtypeStruct((n, hid), jnp.float32),
  )(x, W1, d0, d1)

  agg = _make_agg(n_pad, hid, n_chunks)
  p = agg(g1, src2d, dst2d).reshape(NC, n_pad, hid)

  g2 = pl.pallas_call(
      _tc2_body,
      grid=grid,
      in_specs=[_row_spec(bn, hid), _row_spec(bn, hid), _row_spec(bn, hid),
                _full_spec((1, hid)), _full_spec((hid, hid)),
                _row_spec(bn, 1), _row_spec(bn, 1)],
      out_specs=_row_spec(bn, hid),
      out_shape=jax.ShapeDtypeStruct((n, hid), jnp.float32),
  )(p[0, :n], p[1, :n], g1, b1r, W2, d0, d1)

  q = agg(g2, src2d, dst2d).reshape(NC, n_pad, hid)

  out = pl.pallas_call(
      _tc3_body,
      grid=grid,
      in_specs=[_row_spec(bn, hid), _row_spec(bn, hid), _row_spec(bn, hid),
                _full_spec((1, hid)), _row_spec(bn, 1), _row_spec(bn, 1)],
      out_specs=_row_spec(bn, hid),
      out_shape=jax.ShapeDtypeStruct((n, hid), jnp.float32),
  )(q[0, :n], q[1, :n], g2, b2r, d0, d1)

  return out
